# baseline (speedup 1.0000x reference)
import functools

import jax
import jax.numpy as jnp
from jax import lax
from jax.experimental import pallas as pl
from jax.experimental.pallas import tpu as pltpu
from jax.experimental.pallas import tpu_sc as plsc

B, N, C, P, K = 64, 1024, 96, 1024, 256
NV = N // 16


def _sc_body(conf_hbm, pos_hbm, x2_hbm, pe_hbm,
             pd_out, cs_out, xg_out, pg_out,
             key_v, idx_v, pos_v, pd_v, rix_v, pet_v, xg_v, pg_v, sem):
    i32, f32 = jnp.int32, jnp.float32
    iota16 = lax.iota(i32, 16)

    pltpu.sync_copy(pe_hbm, pet_v)

    wid = lax.axis_index("s") * 2 + lax.axis_index("c")

    def rev16(a):
        return lax.rev(a, (0,))

    def do_batch(t, carry):
        b = wid * 2 + t

        pltpu.sync_copy(conf_hbm.at[b], key_v)
        pltpu.sync_copy(pos_hbm.at[b], pos_v)

        @plsc.parallel_loop(0, NV, unroll=4)
        def _(v):
            idx_v[pl.ds(v * 16, 16)] = iota16 + v * 16

        def sort_pass(s, nv=NV, final=False):
            @plsc.parallel_loop(0, nv, unroll=8)
            def _(v):
                k = key_v[pl.ds(v * 16, 16)]
                x = idx_v[pl.ds(v * 16, 16)]
                ks, xs = plsc.sort_key_val(k, x, descending=True)
                if not final:
                    asc = ((v * 16 // s) % 2) == 1
                    ks = jnp.where(asc, rev16(ks), ks)
                    xs = jnp.where(asc, rev16(xs), xs)
                key_v[pl.ds(v * 16, 16)] = ks
                idx_v[pl.ds(v * 16, 16)] = xs

        sort_pass(16)

        for s in (32, 64, 128, 256, 512, 1024):
            logs = s.bit_length() - 1
            final = s == 1024
            d = s // 2
            while d >= 16:
                dv = d // 16

                @plsc.parallel_loop(0, NV // 2, unroll=8)
                def _(p, dv=dv, logs=logs, final=final):
                    q = p // dv
                    r = p % dv
                    va = q * (2 * dv) + r
                    vb = va + dv
                    ka = key_v[pl.ds(va * 16, 16)]
                    kb = key_v[pl.ds(vb * 16, 16)]
                    ia = idx_v[pl.ds(va * 16, 16)]
                    ib = idx_v[pl.ds(vb * 16, 16)]
                    ct = (ka < kb) | ((ka == kb) & (ia > ib))
                    if final:
                        swap = ct
                    else:
                        asc = (((va * 16) >> logs) & 1) == 1
                        swap = ct != asc
                    key_v[pl.ds(va * 16, 16)] = jnp.where(swap, kb, ka)
                    key_v[pl.ds(vb * 16, 16)] = jnp.where(swap, ka, kb)
                    idx_v[pl.ds(va * 16, 16)] = jnp.where(swap, ib, ia)
                    idx_v[pl.ds(vb * 16, 16)] = jnp.where(swap, ia, ib)

                d //= 2
            sort_pass(s, nv=(18 if final else NV), final=final)

        perm0 = iota16 ^ 1
        even0 = (iota16 % 2) == 0
        perm1 = jnp.clip(((iota16 + 1) ^ 1) - 1, 0, 15)
        first1 = (iota16 % 2) == 1

        def fix_pass(off17, perm, firstmask):
            @plsc.parallel_loop(0, 17, unroll=4)
            def _(v):
                o = off17 + v * 16
                k = key_v[pl.ds(o, 16)]
                x = idx_v[pl.ds(o, 16)]
                kp = jnp.take(k, perm)
                xp = jnp.take(x, perm)
                cond = (k == kp) & ((x > xp) == firstmask)
                idx_v[pl.ds(o, 16)] = jnp.where(cond, xp, x)

        fix_pass(0, perm0, even0)
        fix_pass(8, perm1, first1)
        fix_pass(0, perm0, even0)

        @plsc.parallel_loop(0, K // 16, unroll=4)
        def _(g):
            iv = idx_v[pl.ds(g * 16, 16)]
            pv = plsc.load_gather(pos_v, [iv])
            pd_v[pl.ds(g * 16, 16)] = pv
            rix_v[pl.ds(g * 16, 16)] = iv + b * N

        pltpu.sync_copy(pd_v, pd_out.at[b])
        pltpu.sync_copy(key_v.at[pl.ds(0, K)], cs_out.at[b])

        for ch in range(2):
            idx_ref = rix_v.at[pl.ds(ch * 128, 128)]
            pltpu.async_copy(x2_hbm.at[idx_ref], xg_v, sem).wait()
            pltpu.sync_copy(xg_v, xg_out.at[pl.ds(b * K + ch * 128, 128)])

        for ch in range(4):
            @plsc.parallel_loop(0, C, unroll=4)
            def _(j, ch=ch):
                for g4 in range(4):
                    kk = ch * 64 + g4 * 16
                    pb = pd_v[pl.ds(kk, 16)] * C + j
                    v = plsc.load_gather(pet_v, [pb])
                    plsc.store_scatter(pg_v, [g4 * 16 + iota16,
                                              jnp.broadcast_to(j, (16,))], v)
            pltpu.sync_copy(pg_v, pg_out.at[pl.ds(b * K + ch * 64, 64)])
        return carry

    lax.fori_loop(0, 2, do_batch, 0)


_sc_topk = functools.partial(
    pl.kernel,
    mesh=plsc.VectorSubcoreMesh(core_axis_name="c", subcore_axis_name="s"),
    compiler_params=pltpu.CompilerParams(needs_layout_passes=False, use_tc_tiling_on_sc=False),
    out_type=(
        jax.ShapeDtypeStruct((B, K), jnp.int32),
        jax.ShapeDtypeStruct((B, K), jnp.float32),
        jax.ShapeDtypeStruct((B * K, C), jnp.float32),
        jax.ShapeDtypeStruct((B * K, C), jnp.float32),
    ),
    scratch_types=[
        pltpu.VMEM((N,), jnp.float32),
        pltpu.VMEM((N,), jnp.int32),
        pltpu.VMEM((N,), jnp.int32),
        pltpu.VMEM((K,), jnp.int32),
        pltpu.VMEM((K,), jnp.int32),
        pltpu.VMEM((P * C,), jnp.float32),
        pltpu.VMEM((128, C), jnp.float32),
        pltpu.VMEM((64, C), jnp.float32),
        pltpu.SemaphoreType.DMA,
    ],
)(_sc_body)


def _tc_body(xg_ref, pg_ref, cs_ref, gamma_ref, beta_ref, xd_ref):
    x = xg_ref[0]
    mean = jnp.mean(x, axis=-1, keepdims=True)
    xc = x - mean
    var = jnp.mean(xc * xc, axis=-1, keepdims=True)
    xn = xc / jnp.sqrt(var + 1e-5) * gamma_ref[0] + beta_ref[0]
    xd_ref[0] = xn * cs_ref[0] + pg_ref[0]


def kernel(x, pos, pos_embed, gamma, beta, W_conf, b_conf):
    mean = jnp.mean(x, axis=-1, keepdims=True)
    xc = x - mean
    var = jnp.mean(xc * xc, axis=-1, keepdims=True)
    xn = xc / jnp.sqrt(var + 1e-5) * gamma + beta
    c = xn @ W_conf + b_conf
    conf = jax.nn.softmax(c, axis=1) * N

    conf2 = conf.reshape(B, N)
    x2 = x.reshape(B * N, C)
    pe1 = pos_embed.reshape(P * C)

    pos_down, conf_sel, xg, pg = _sc_topk(conf2, pos, x2, pe1)

    xg3 = xg.reshape(B, K, C)
    pg3 = pg.reshape(B, K, C)
    cs3 = conf_sel.reshape(B, K, 1)
    gamma2 = gamma.reshape(1, C)
    beta2 = beta.reshape(1, C)

    x_down = pl.pallas_call(
        _tc_body,
        grid=(B,),
        in_specs=[
            pl.BlockSpec((1, K, C), lambda i: (i, 0, 0)),
            pl.BlockSpec((1, K, C), lambda i: (i, 0, 0)),
            pl.BlockSpec((1, K, 1), lambda i: (i, 0, 0)),
            pl.BlockSpec((1, C), lambda i: (0, 0)),
            pl.BlockSpec((1, C), lambda i: (0, 0)),
        ],
        out_specs=pl.BlockSpec((1, K, C), lambda i: (i, 0, 0)),
        out_shape=jax.ShapeDtypeStruct((B, K, C), jnp.float32),
    )(xg3, pg3, cs3, gamma2, beta2)
    return (x_down, pos_down)
